# Initial kernel scaffold; baseline (speedup 1.0000x reference)
import functools

import jax
import jax.numpy as jnp
from jax import lax
from jax.experimental import pallas as pl
from jax.experimental.pallas import tpu as pltpu
from jax.experimental.pallas import tpu_sc as plsc

VOCAB1 = 100001
D = 32
B = 16384

_info = plsc.get_sparse_core_info()
_NC, _NS = _info.num_cores, _info.num_subcores
_NW = _NC * _NS
_BPW = B // _NW

_mesh = plsc.VectorSubcoreMesh(core_axis_name="c", subcore_axis_name="s")


@functools.partial(
    pl.kernel,
    mesh=_mesh,
    out_type=jax.ShapeDtypeStruct((B, D), jnp.float32),
    scratch_types=[
        pltpu.VMEM((_BPW,), jnp.int32),
        pltpu.VMEM((_BPW, D), jnp.float32),
        pltpu.SemaphoreType.DMA,
    ],
)
def _sc_gather(table_hbm, idx_hbm, out_hbm, idx_v, rows_v, sem):
    wid = lax.axis_index("s") * _NC + lax.axis_index("c")
    base = wid * _BPW
    pltpu.sync_copy(idx_hbm.at[pl.ds(base, _BPW)], idx_v)
    pltpu.async_copy(table_hbm.at[idx_v], rows_v, sem).wait()
    pltpu.sync_copy(rows_v, out_hbm.at[pl.ds(base, _BPW)])


def _mlp_body(feat_ref, w1_ref, b1_ref, w2_ref, b2_ref, w3_ref, b3_ref,
              out_ref):
    h = jnp.dot(feat_ref[...], w1_ref[...],
                preferred_element_type=jnp.float32) + b1_ref[...]
    h = jnp.maximum(h, 0.0)
    h = jnp.dot(h, w2_ref[...], preferred_element_type=jnp.float32) + b2_ref[...]
    h = jnp.maximum(h, 0.0)
    out_ref[...] = jnp.dot(h, w3_ref[...],
                           preferred_element_type=jnp.float32) + b3_ref[...]


_NB = 2048


def _mlp(feat, W1, b1, W2, b2, W3, b3):
    full = lambda shape: pl.BlockSpec(shape, lambda i: (0,) * len(shape))
    return pl.pallas_call(
        _mlp_body,
        grid=(B // _NB,),
        in_specs=[
            pl.BlockSpec((_NB, D), lambda i: (i, 0)),
            full((D, 256)),
            full((1, 256)),
            full((256, 128)),
            full((1, 128)),
            full((128, 64)),
            full((1, 64)),
        ],
        out_specs=pl.BlockSpec((_NB, 64), lambda i: (i, 0)),
        out_shape=jax.ShapeDtypeStruct((B, 64), jnp.float32),
        compiler_params=pltpu.CompilerParams(
            dimension_semantics=("parallel",)),
    )(feat, W1, b1, W2, b2, W3, b3)


def kernel(AuthorId, table, W1, b1, W2, b2, W3, b3):
    idx = AuthorId.astype(jnp.int32)
    feat = _sc_gather(table, idx)
    return _mlp(feat, W1, b1.reshape(1, -1), W2, b2.reshape(1, -1),
                W3, b3.reshape(1, -1))


# speedup vs baseline: 1.5692x; 1.5692x over previous
import functools

import jax
import jax.numpy as jnp
from jax import lax
from jax.experimental import pallas as pl
from jax.experimental.pallas import tpu as pltpu
from jax.experimental.pallas import tpu_sc as plsc

VROW = 100001
D = 32
B = 16384

_info = plsc.get_sparse_core_info()
_NC, _NS = _info.num_cores, _info.num_subcores
_NW = _NC * _NS
assert _NW == D

_ROWWIN = 100008
_CHUNK = 4096

_mesh = plsc.VectorSubcoreMesh(core_axis_name="c", subcore_axis_name="s")


@functools.partial(
    pl.kernel,
    mesh=_mesh,
    out_type=jax.ShapeDtypeStruct((D * B,), jnp.float32),
    scratch_types=[
        pltpu.VMEM((_ROWWIN,), jnp.float32),
        pltpu.VMEM((_CHUNK,), jnp.int32),
        pltpu.VMEM((_CHUNK,), jnp.float32),
    ],
    compiler_params=pltpu.CompilerParams(use_tc_tiling_on_sc=False,
                                         needs_layout_passes=False),
)
def _sc_gather_t(tflat_hbm, idx_hbm, out_hbm, row_v, idx_v, val_v):
    wid = lax.axis_index("s") * _NC + lax.axis_index("c")
    r = lax.rem(wid * VROW, 8)
    start = pl.multiple_of(wid * VROW - r, 8)
    pltpu.sync_copy(tflat_hbm.at[pl.ds(start, _ROWWIN)], row_v)
    for c in range(B // _CHUNK):
        pltpu.sync_copy(idx_hbm.at[pl.ds(c * _CHUNK, _CHUNK)], idx_v)

        def body(i, carry):
            pos = idx_v[pl.ds(i * 16, 16)] + r
            val_v[pl.ds(i * 16, 16)] = plsc.load_gather(row_v, [pos])
            return carry

        lax.fori_loop(0, _CHUNK // 16, body, 0)
        pltpu.sync_copy(val_v, out_hbm.at[pl.ds(wid * B + c * _CHUNK, _CHUNK)])


_NB = 2048

_CONTRACT00 = (((0,), (0,)), ((), ()))


def _mlp_t_body(featT_ref, w1_ref, b1_ref, w2_ref, b2_ref, w3_ref, b3_ref,
                out_ref):
    h = lax.dot_general(w1_ref[...], featT_ref[...], _CONTRACT00,
                        preferred_element_type=jnp.float32) + b1_ref[...]
    h = jnp.maximum(h, 0.0)
    h = lax.dot_general(w2_ref[...], h, _CONTRACT00,
                        preferred_element_type=jnp.float32) + b2_ref[...]
    h = jnp.maximum(h, 0.0)
    out_ref[...] = lax.dot_general(w3_ref[...], h, _CONTRACT00,
                                   preferred_element_type=jnp.float32) + b3_ref[...]


def _mlp_t(featT, W1, b1, W2, b2, W3, b3):
    full = lambda shape: pl.BlockSpec(shape, lambda i: (0,) * len(shape))
    return pl.pallas_call(
        _mlp_t_body,
        grid=(B // _NB,),
        in_specs=[
            pl.BlockSpec((D, _NB), lambda i: (0, i)),
            full((D, 256)),
            full((256, 1)),
            full((256, 128)),
            full((128, 1)),
            full((128, 64)),
            full((64, 1)),
        ],
        out_specs=pl.BlockSpec((64, _NB), lambda i: (0, i)),
        out_shape=jax.ShapeDtypeStruct((64, B), jnp.float32),
        compiler_params=pltpu.CompilerParams(
            dimension_semantics=("parallel",)),
    )(featT, W1, b1, W2, b2, W3, b3)


def kernel(AuthorId, table, W1, b1, W2, b2, W3, b3):
    idx = AuthorId.astype(jnp.int32)
    tflat = table.T.reshape(-1)
    featT = _sc_gather_t(tflat, idx).reshape(D, B)
    outT = _mlp_t(featT, W1, b1.reshape(-1, 1), W2, b2.reshape(-1, 1),
                  W3, b3.reshape(-1, 1))
    return outT.T


# speedup vs baseline: 1.6476x; 1.0500x over previous
import functools

import jax
import jax.numpy as jnp
from jax import lax
from jax.experimental import pallas as pl
from jax.experimental.pallas import tpu as pltpu
from jax.experimental.pallas import tpu_sc as plsc

VROW = 100001
D = 32
B = 16384

_info = plsc.get_sparse_core_info()
_NC, _NS = _info.num_cores, _info.num_subcores
_NW = _NC * _NS
assert _NW == D

_ROWWIN = 100008
_CHUNK = 4096

_mesh = plsc.VectorSubcoreMesh(core_axis_name="c", subcore_axis_name="s")


@functools.partial(
    pl.kernel,
    mesh=_mesh,
    out_type=jax.ShapeDtypeStruct((D * B,), jnp.float32),
    scratch_types=[
        pltpu.VMEM((_ROWWIN,), jnp.float32),
        pltpu.VMEM((_CHUNK,), jnp.int32),
        pltpu.VMEM((_CHUNK,), jnp.float32),
    ],
    compiler_params=pltpu.CompilerParams(use_tc_tiling_on_sc=False,
                                         needs_layout_passes=False),
)
def _sc_gather_t(tflat_hbm, idx_hbm, out_hbm, row_v, idx_v, val_v):
    wid = lax.axis_index("s") * _NC + lax.axis_index("c")
    r = lax.rem(wid * VROW, 8)
    start = pl.multiple_of(wid * VROW - r, 8)
    pltpu.sync_copy(tflat_hbm.at[pl.ds(start, _ROWWIN)], row_v)
    for c in range(B // _CHUNK):
        pltpu.sync_copy(idx_hbm.at[pl.ds(c * _CHUNK, _CHUNK)], idx_v)

        @plsc.parallel_loop(0, _CHUNK, step=16, unroll=8)
        def body(i):
            pos = idx_v[pl.ds(i, 16)] + r
            val_v[pl.ds(i, 16)] = plsc.load_gather(row_v, [pos])

        pltpu.sync_copy(val_v, out_hbm.at[pl.ds(wid * B + c * _CHUNK, _CHUNK)])


_NB = 2048

_CONTRACT00 = (((0,), (0,)), ((), ()))


def _mlp_t_body(featT_ref, w1_ref, b1_ref, w2_ref, b2_ref, w3_ref, b3_ref,
                out_ref):
    h = lax.dot_general(w1_ref[...], featT_ref[...], _CONTRACT00,
                        preferred_element_type=jnp.float32) + b1_ref[...]
    h = jnp.maximum(h, 0.0)
    h = lax.dot_general(w2_ref[...], h, _CONTRACT00,
                        preferred_element_type=jnp.float32) + b2_ref[...]
    h = jnp.maximum(h, 0.0)
    out_ref[...] = lax.dot_general(w3_ref[...], h, _CONTRACT00,
                                   preferred_element_type=jnp.float32) + b3_ref[...]


def _mlp_t(featT, W1, b1, W2, b2, W3, b3):
    full = lambda shape: pl.BlockSpec(shape, lambda i: (0,) * len(shape))
    return pl.pallas_call(
        _mlp_t_body,
        grid=(B // _NB,),
        in_specs=[
            pl.BlockSpec((D, _NB), lambda i: (0, i)),
            full((D, 256)),
            full((256, 1)),
            full((256, 128)),
            full((128, 1)),
            full((128, 64)),
            full((64, 1)),
        ],
        out_specs=pl.BlockSpec((64, _NB), lambda i: (0, i)),
        out_shape=jax.ShapeDtypeStruct((64, B), jnp.float32),
        compiler_params=pltpu.CompilerParams(
            dimension_semantics=("parallel",)),
    )(featT, W1, b1, W2, b2, W3, b3)


def kernel(AuthorId, table, W1, b1, W2, b2, W3, b3):
    idx = AuthorId.astype(jnp.int32)
    tflat = table.T.reshape(-1)
    featT = _sc_gather_t(tflat, idx).reshape(D, B)
    outT = _mlp_t(featT, W1, b1.reshape(-1, 1), W2, b2.reshape(-1, 1),
                  W3, b3.reshape(-1, 1))
    return outT.T


# speedup vs baseline: 2.3337x; 1.4164x over previous
import functools

import jax
import jax.numpy as jnp
from jax import lax
from jax.experimental import pallas as pl
from jax.experimental.pallas import tpu as pltpu
from jax.experimental.pallas import tpu_sc as plsc

VROW = 100001
D = 32
B = 16384

_info = plsc.get_sparse_core_info()
_NC, _NS = _info.num_cores, _info.num_subcores
_NW = _NC * _NS
assert _NW == D

_CHUNK = 4096

_mesh = plsc.VectorSubcoreMesh(core_axis_name="c", subcore_axis_name="s")


@functools.partial(
    pl.kernel,
    mesh=_mesh,
    out_type=jax.ShapeDtypeStruct((4, 8, B), jnp.float32),
    scratch_types=[
        pltpu.VMEM((VROW,), jnp.float32),
        pltpu.VMEM((_CHUNK,), jnp.int32),
        pltpu.VMEM((_CHUNK,), jnp.float32),
    ],
    compiler_params=pltpu.CompilerParams(use_tc_tiling_on_sc=True,
                                         needs_layout_passes=False),
)
def _sc_gather_t(t3_hbm, idx_hbm, out_hbm, row_v, idx_v, val_v):
    wid = lax.axis_index("s") * _NC + lax.axis_index("c")
    q = wid // 8
    s = wid % 8
    pltpu.sync_copy(t3_hbm.at[q, s], row_v)
    for c in range(B // _CHUNK):
        pltpu.sync_copy(idx_hbm.at[pl.ds(c * _CHUNK, _CHUNK)], idx_v)

        @plsc.parallel_loop(0, _CHUNK, step=16, unroll=8)
        def body(i):
            pos = idx_v[pl.ds(i, 16)]
            val_v[pl.ds(i, 16)] = plsc.load_gather(row_v, [pos])

        pltpu.sync_copy(val_v, out_hbm.at[q, s, pl.ds(c * _CHUNK, _CHUNK)])


_NB = 2048

_CONTRACT00 = (((0,), (0,)), ((), ()))


def _mlp_t_body(featT_ref, w1_ref, b1_ref, w2_ref, b2_ref, w3_ref, b3_ref,
                out_ref):
    h = lax.dot_general(w1_ref[...], featT_ref[...], _CONTRACT00,
                        preferred_element_type=jnp.float32) + b1_ref[...]
    h = jnp.maximum(h, 0.0)
    h = lax.dot_general(w2_ref[...], h, _CONTRACT00,
                        preferred_element_type=jnp.float32) + b2_ref[...]
    h = jnp.maximum(h, 0.0)
    out_ref[...] = lax.dot_general(w3_ref[...], h, _CONTRACT00,
                                   preferred_element_type=jnp.float32) + b3_ref[...]


def _mlp_t(featT, W1, b1, W2, b2, W3, b3):
    full = lambda shape: pl.BlockSpec(shape, lambda i: (0,) * len(shape))
    return pl.pallas_call(
        _mlp_t_body,
        grid=(B // _NB,),
        in_specs=[
            pl.BlockSpec((D, _NB), lambda i: (0, i)),
            full((D, 256)),
            full((256, 1)),
            full((256, 128)),
            full((128, 1)),
            full((128, 64)),
            full((64, 1)),
        ],
        out_specs=pl.BlockSpec((64, _NB), lambda i: (0, i)),
        out_shape=jax.ShapeDtypeStruct((64, B), jnp.float32),
        compiler_params=pltpu.CompilerParams(
            dimension_semantics=("parallel",)),
    )(featT, W1, b1, W2, b2, W3, b3)


def kernel(AuthorId, table, W1, b1, W2, b2, W3, b3):
    idx = AuthorId.astype(jnp.int32)
    t3 = table.T.reshape(4, 8, VROW)
    featT = _sc_gather_t(t3, idx).reshape(D, B)
    outT = _mlp_t(featT, W1, b1.reshape(-1, 1), W2, b2.reshape(-1, 1),
                  W3, b3.reshape(-1, 1))
    return outT.T


# speedup vs baseline: 2.5279x; 1.0832x over previous
import functools

import jax
import jax.numpy as jnp
from jax import lax
from jax.experimental import pallas as pl
from jax.experimental.pallas import tpu as pltpu
from jax.experimental.pallas import tpu_sc as plsc

VROW = 100001
D = 32
B = 16384

_info = plsc.get_sparse_core_info()
_NC, _NS = _info.num_cores, _info.num_subcores
_NW = _NC * _NS
assert _NW == D

_CHUNK = 4096

_mesh = plsc.VectorSubcoreMesh(core_axis_name="c", subcore_axis_name="s")


@functools.partial(
    pl.kernel,
    mesh=_mesh,
    out_type=jax.ShapeDtypeStruct((4, 8, B), jnp.float32),
    scratch_types=[
        pltpu.VMEM((VROW,), jnp.float32),
        pltpu.VMEM((B,), jnp.int32),
        pltpu.VMEM((_CHUNK,), jnp.float32),
        pltpu.VMEM((_CHUNK,), jnp.float32),
        pltpu.SemaphoreType.DMA,
        pltpu.SemaphoreType.DMA,
        pltpu.SemaphoreType.DMA,
        pltpu.SemaphoreType.DMA,
    ],
    compiler_params=pltpu.CompilerParams(use_tc_tiling_on_sc=True,
                                         needs_layout_passes=False),
)
def _sc_gather_t(t3_hbm, idx_hbm, out_hbm, row_v, idx_v, val0, val1,
                 sem_row, sem_idx, sem_o0, sem_o1):
    wid = lax.axis_index("s") * _NC + lax.axis_index("c")
    q = wid // 8
    s = wid % 8
    vals = (val0, val1)
    osems = (sem_o0, sem_o1)
    with jax.named_scope("stage"):
        h_idx = pltpu.async_copy(idx_hbm, idx_v, sem_idx)
        h_row = pltpu.async_copy(t3_hbm.at[q, s], row_v, sem_row)
        h_idx.wait()
        h_row.wait()
    outs = [None, None]
    for c in range(B // _CHUNK):
        buf = vals[c % 2]
        if outs[c % 2] is not None:
            outs[c % 2].wait()

        with jax.named_scope("resolve"):
            @plsc.parallel_loop(0, _CHUNK, step=16, unroll=8)
            def body(i):
                pos = idx_v[pl.ds(c * _CHUNK + i, 16)]
                buf[pl.ds(i, 16)] = plsc.load_gather(row_v, [pos])

        outs[c % 2] = pltpu.async_copy(
            buf, out_hbm.at[q, s, pl.ds(c * _CHUNK, _CHUNK)], osems[c % 2])
    outs[0].wait()
    outs[1].wait()


_NB = 2048

_CONTRACT00 = (((0,), (0,)), ((), ()))


def _mlp_t_body(featT_ref, w1_ref, b1_ref, w2_ref, b2_ref, w3_ref, b3_ref,
                out_ref):
    h = lax.dot_general(w1_ref[...], featT_ref[...], _CONTRACT00,
                        preferred_element_type=jnp.float32) + b1_ref[...]
    h = jnp.maximum(h, 0.0)
    h = lax.dot_general(w2_ref[...], h, _CONTRACT00,
                        preferred_element_type=jnp.float32) + b2_ref[...]
    h = jnp.maximum(h, 0.0)
    out_ref[...] = lax.dot_general(w3_ref[...], h, _CONTRACT00,
                                   preferred_element_type=jnp.float32) + b3_ref[...]


def _mlp_t(featT, W1, b1, W2, b2, W3, b3):
    full = lambda shape: pl.BlockSpec(shape, lambda i: (0,) * len(shape))
    return pl.pallas_call(
        _mlp_t_body,
        grid=(B // _NB,),
        in_specs=[
            pl.BlockSpec((D, _NB), lambda i: (0, i)),
            full((D, 256)),
            full((256, 1)),
            full((256, 128)),
            full((128, 1)),
            full((128, 64)),
            full((64, 1)),
        ],
        out_specs=pl.BlockSpec((64, _NB), lambda i: (0, i)),
        out_shape=jax.ShapeDtypeStruct((64, B), jnp.float32),
        compiler_params=pltpu.CompilerParams(
            dimension_semantics=("parallel",)),
    )(featT, W1, b1, W2, b2, W3, b3)


def kernel(AuthorId, table, W1, b1, W2, b2, W3, b3):
    idx = AuthorId.astype(jnp.int32)
    t3 = table.T.reshape(4, 8, VROW)
    featT = _sc_gather_t(t3, idx).reshape(D, B)
    outT = _mlp_t(featT, W1, b1.reshape(-1, 1), W2, b2.reshape(-1, 1),
                  W3, b3.reshape(-1, 1))
    return outT.T


# speedup vs baseline: 2.6848x; 1.0621x over previous
import functools

import jax
import jax.numpy as jnp
from jax import lax
from jax.experimental import pallas as pl
from jax.experimental.pallas import tpu as pltpu
from jax.experimental.pallas import tpu_sc as plsc

VROW = 100001
D = 32
B = 16384

_info = plsc.get_sparse_core_info()
_NC, _NS = _info.num_cores, _info.num_subcores
_NW = _NC * _NS
assert _NW == D

_CHUNK = 4096

_mesh = plsc.VectorSubcoreMesh(core_axis_name="c", subcore_axis_name="s")


@functools.partial(
    pl.kernel,
    mesh=_mesh,
    out_type=jax.ShapeDtypeStruct((4, 8, B), jnp.float32),
    scratch_types=[
        pltpu.VMEM((VROW,), jnp.float32),
        pltpu.VMEM((B,), jnp.int32),
        pltpu.VMEM((_CHUNK,), jnp.float32),
        pltpu.VMEM((_CHUNK,), jnp.float32),
        pltpu.SemaphoreType.DMA,
        pltpu.SemaphoreType.DMA,
        pltpu.SemaphoreType.DMA,
        pltpu.SemaphoreType.DMA,
    ],
    compiler_params=pltpu.CompilerParams(use_tc_tiling_on_sc=True,
                                         needs_layout_passes=False),
)
def _sc_gather_t(t3_hbm, idx_hbm, out_hbm, row_v, idx_v, val0, val1,
                 sem_row, sem_idx, sem_o0, sem_o1):
    wid = lax.axis_index("s") * _NC + lax.axis_index("c")
    q = wid // 8
    s = wid % 8
    vals = (val0, val1)
    osems = (sem_o0, sem_o1)
    with jax.named_scope("stage"):
        h_idx = pltpu.async_copy(idx_hbm, idx_v, sem_idx)
        h_row = pltpu.async_copy(t3_hbm.at[q, s], row_v, sem_row)
        h_idx.wait()
        h_row.wait()
    outs = [None, None]
    for c in range(B // _CHUNK):
        buf = vals[c % 2]
        if outs[c % 2] is not None:
            outs[c % 2].wait()

        with jax.named_scope("resolve"):
            @plsc.parallel_loop(0, _CHUNK, step=16, unroll=8)
            def body(i):
                pos = idx_v[pl.ds(c * _CHUNK + i, 16)]
                buf[pl.ds(i, 16)] = plsc.load_gather(row_v, [pos])

        outs[c % 2] = pltpu.async_copy(
            buf, out_hbm.at[q, s, pl.ds(c * _CHUNK, _CHUNK)], osems[c % 2])
    outs[0].wait()
    outs[1].wait()


_NB = 4096

_CONTRACT00 = (((0,), (0,)), ((), ()))


def _mlp_t_body(featT_ref, w1_ref, b1_ref, w2_ref, b2_ref, w3_ref, b3_ref,
                out_ref):
    h = lax.dot_general(w1_ref[...], featT_ref[...], _CONTRACT00,
                        preferred_element_type=jnp.float32) + b1_ref[...]
    h = jnp.maximum(h, 0.0)
    h = lax.dot_general(w2_ref[...], h, _CONTRACT00,
                        preferred_element_type=jnp.float32) + b2_ref[...]
    h = jnp.maximum(h, 0.0)
    out_ref[...] = lax.dot_general(w3_ref[...], h, _CONTRACT00,
                                   preferred_element_type=jnp.float32) + b3_ref[...]


def _mlp_t(featT, W1, b1, W2, b2, W3, b3):
    full = lambda shape: pl.BlockSpec(shape, lambda i: (0,) * len(shape))
    return pl.pallas_call(
        _mlp_t_body,
        grid=(B // _NB,),
        in_specs=[
            pl.BlockSpec((D, _NB), lambda i: (0, i)),
            full((D, 256)),
            full((256, 1)),
            full((256, 128)),
            full((128, 1)),
            full((128, 64)),
            full((64, 1)),
        ],
        out_specs=pl.BlockSpec((64, _NB), lambda i: (0, i)),
        out_shape=jax.ShapeDtypeStruct((64, B), jnp.float32),
        compiler_params=pltpu.CompilerParams(
            dimension_semantics=("parallel",)),
    )(featT, W1, b1, W2, b2, W3, b3)


def kernel(AuthorId, table, W1, b1, W2, b2, W3, b3):
    idx = AuthorId.astype(jnp.int32)
    t3 = table.T.reshape(4, 8, VROW)
    featT = _sc_gather_t(t3, idx).reshape(D, B)
    outT = _mlp_t(featT, W1, b1.reshape(-1, 1), W2, b2.reshape(-1, 1),
                  W3, b3.reshape(-1, 1))
    return outT.T


# speedup vs baseline: 2.7088x; 1.0089x over previous
import functools

import jax
import jax.numpy as jnp
from jax import lax
from jax.experimental import pallas as pl
from jax.experimental.pallas import tpu as pltpu
from jax.experimental.pallas import tpu_sc as plsc

VROW = 100001
D = 32
B = 16384

_info = plsc.get_sparse_core_info()
_NC, _NS = _info.num_cores, _info.num_subcores
_NW = _NC * _NS
assert _NW == D

_CHUNK = 4096

_mesh = plsc.VectorSubcoreMesh(core_axis_name="c", subcore_axis_name="s")


@functools.partial(
    pl.kernel,
    mesh=_mesh,
    out_type=jax.ShapeDtypeStruct((4, 8, B), jnp.float32),
    scratch_types=[
        pltpu.VMEM((VROW,), jnp.float32),
        pltpu.VMEM((B,), jnp.int32),
        pltpu.VMEM((_CHUNK,), jnp.float32),
        pltpu.VMEM((_CHUNK,), jnp.float32),
        pltpu.SemaphoreType.DMA,
        pltpu.SemaphoreType.DMA,
        pltpu.SemaphoreType.DMA,
        pltpu.SemaphoreType.DMA,
    ],
    compiler_params=pltpu.CompilerParams(use_tc_tiling_on_sc=True,
                                         needs_layout_passes=False),
)
def _sc_gather_t(t3_hbm, idx_hbm, out_hbm, row_v, idx_v, val0, val1,
                 sem_row, sem_idx, sem_o0, sem_o1):
    wid = lax.axis_index("s") * _NC + lax.axis_index("c")
    q = wid // 8
    s = wid % 8
    vals = (val0, val1)
    osems = (sem_o0, sem_o1)
    with jax.named_scope("stage"):
        h_idx = pltpu.async_copy(idx_hbm, idx_v, sem_idx)
        h_row = pltpu.async_copy(t3_hbm.at[q, s], row_v, sem_row)
        h_idx.wait()
        h_row.wait()
    outs = [None, None]
    for c in range(B // _CHUNK):
        buf = vals[c % 2]
        if outs[c % 2] is not None:
            outs[c % 2].wait()

        with jax.named_scope("resolve"):
            @plsc.parallel_loop(0, _CHUNK, step=16, unroll=8)
            def body(i):
                pos = idx_v[pl.ds(c * _CHUNK + i, 16)]
                buf[pl.ds(i, 16)] = plsc.load_gather(row_v, [pos])

        outs[c % 2] = pltpu.async_copy(
            buf, out_hbm.at[q, s, pl.ds(c * _CHUNK, _CHUNK)], osems[c % 2])
    outs[0].wait()
    outs[1].wait()


_NB = 8192

_CONTRACT00 = (((0,), (0,)), ((), ()))


def _mlp_t_body(featT_ref, w1_ref, b1_ref, w2_ref, b2_ref, w3_ref, b3_ref,
                out_ref):
    h = lax.dot_general(w1_ref[...], featT_ref[...], _CONTRACT00,
                        preferred_element_type=jnp.float32) + b1_ref[...]
    h = jnp.maximum(h, 0.0)
    h = lax.dot_general(w2_ref[...], h, _CONTRACT00,
                        preferred_element_type=jnp.float32) + b2_ref[...]
    h = jnp.maximum(h, 0.0)
    out_ref[...] = lax.dot_general(w3_ref[...], h, _CONTRACT00,
                                   preferred_element_type=jnp.float32) + b3_ref[...]


def _mlp_t(featT, W1, b1, W2, b2, W3, b3):
    full = lambda shape: pl.BlockSpec(shape, lambda i: (0,) * len(shape))
    return pl.pallas_call(
        _mlp_t_body,
        grid=(B // _NB,),
        in_specs=[
            pl.BlockSpec((D, _NB), lambda i: (0, i)),
            full((D, 256)),
            full((256, 1)),
            full((256, 128)),
            full((128, 1)),
            full((128, 64)),
            full((64, 1)),
        ],
        out_specs=pl.BlockSpec((64, _NB), lambda i: (0, i)),
        out_shape=jax.ShapeDtypeStruct((64, B), jnp.float32),
        compiler_params=pltpu.CompilerParams(
            dimension_semantics=("parallel",)),
    )(featT, W1, b1, W2, b2, W3, b3)


def kernel(AuthorId, table, W1, b1, W2, b2, W3, b3):
    idx = AuthorId.astype(jnp.int32)
    t3 = table.T.reshape(4, 8, VROW)
    featT = _sc_gather_t(t3, idx).reshape(D, B)
    outT = _mlp_t(featT, W1, b1.reshape(-1, 1), W2, b2.reshape(-1, 1),
                  W3, b3.reshape(-1, 1))
    return outT.T
